# baseline (speedup 1.0000x reference)
import functools

import jax
import jax.numpy as jnp
from jax import lax
from jax.experimental import pallas as pl
from jax.experimental.pallas import tpu as pltpu
from jax.experimental.pallas import tpu_sc as plsc

N = 26
EMB = 32
OUT = 64
H1 = 256
H2 = 128
BATCH = 16384

NC = 2
NS = 16
NW = NC * NS

B_FLAT = BATCH * N
ROWS_PER_W = B_FLAT // NW
IDX_CHUNK = 128
IDX_ROWS = ROWS_PER_W // IDX_CHUNK
GATHERS_PER_STEP = 8
STEPS = IDX_ROWS // GATHERS_PER_STEP
STEP_ROWS = GATHERS_PER_STEP * IDX_CHUNK
NBUF = 4

TAB128 = 250000
OUT128_ROWS = B_FLAT * EMB // 128
OUT128_PER_W = OUT128_ROWS // NW
OUT128_PER_STEP = STEP_ROWS * EMB // 128


NCH = 7812
TAIL_ROWS = 64


def _table_convert(tt, tail128):
    mesh = plsc.VectorSubcoreMesh(
        core_axis_name="c", subcore_axis_name="s", num_cores=NC,
        num_subcores=NS)

    @functools.partial(
        pl.kernel,
        out_type=jax.ShapeDtypeStruct((TAB128, 128), jnp.float32),
        mesh=mesh,
        compiler_params=pltpu.CompilerParams(use_tc_tiling_on_sc=True,
                                             needs_layout_passes=False),
        scratch_types=[
            pltpu.VMEM((2, EMB, 128), jnp.float32),
            pltpu.VMEM((2, EMB, 128), jnp.float32),
            pltpu.SemaphoreType.DMA,
            pltpu.SemaphoreType.DMA,
            pltpu.SemaphoreType.DMA,
            pltpu.SemaphoreType.DMA,
        ],
    )
    def k(tt_hbm, tail_hbm, out_hbm, inb, outb, g0, g1, w0, w1):
        wid = lax.axis_index("s") * NC + lax.axis_index("c")
        rows_lo = lax.broadcasted_iota(jnp.int32, (16,), 0)
        rows_hi = rows_lo + 16

        def fire_in(t, b, sem):
            pltpu.async_copy(tt_hbm.at[:, pl.ds(t * 128, 128)],
                             inb.at[b], sem)

        def wait_in(t, b, sem):
            pltpu.make_async_copy(tt_hbm.at[:, pl.ds(t * 128, 128)],
                                  inb.at[b], sem).wait()

        def transpose(b):
            def body(c2, carry):
                for cc in range(4):
                    col = jnp.broadcast_to(c2 * 4 + cc, (16,))
                    lo = plsc.load_gather(inb.at[b], [rows_lo, col])
                    hi = plsc.load_gather(inb.at[b], [rows_hi, col])
                    outb[b, c2, pl.ds(cc * 32, 16)] = lo
                    outb[b, c2, pl.ds(cc * 32 + 16, 16)] = hi
                return carry

            lax.fori_loop(0, 32, body, 0)

        def write_out(t, b, sem):
            pltpu.async_copy(outb.at[b],
                             out_hbm.at[pl.ds(t * EMB, EMB)], sem)

        def drain_out(b, sem):
            pltpu.make_async_copy(outb.at[b],
                                  out_hbm.at[pl.ds(0, EMB)], sem).wait()

        @pl.when(wid < NCH)
        def _():
            fire_in(wid, 0, g0)

        def pair(v, carry):
            t0 = wid + 64 * v
            t1 = t0 + 32
            t2 = t0 + 64

            @pl.when(t0 < NCH)
            def _():
                @pl.when(t1 < NCH)
                def _():
                    fire_in(t1, 1, g1)

                wait_in(t0, 0, g0)

                @pl.when(v > 0)
                def _():
                    drain_out(0, w0)

                transpose(0)
                write_out(t0, 0, w0)

            @pl.when(t1 < NCH)
            def _():
                @pl.when(t2 < NCH)
                def _():
                    fire_in(t2, 0, g0)

                wait_in(t1, 1, g1)

                @pl.when(v > 0)
                def _():
                    drain_out(1, w1)

                transpose(1)
                write_out(t1, 1, w1)

            return carry

        lax.fori_loop(0, (NCH + 63) // 64, pair, 0)
        drain_out(0, w0)
        drain_out(1, w1)

        @pl.when(wid == 0)
        def _():
            pltpu.sync_copy(tail_hbm, outb.at[0, pl.ds(0, 16)])
            pltpu.sync_copy(outb.at[0, pl.ds(0, 16)],
                            out_hbm.at[pl.ds(TAB128 - 16, 16)])

    return k(tt, tail128)


def _sc_gather(table128, idx):
    mesh = plsc.VectorSubcoreMesh(
        core_axis_name="c", subcore_axis_name="s", num_cores=NC,
        num_subcores=NS)

    @functools.partial(
        pl.kernel,
        out_type=jax.ShapeDtypeStruct((OUT128_ROWS, 128), jnp.float32),
        mesh=mesh,
        compiler_params=pltpu.CompilerParams(use_tc_tiling_on_sc=True),
        scratch_types=[
            pltpu.VMEM((IDX_ROWS, IDX_CHUNK), jnp.int32),
            pltpu.VMEM((IDX_ROWS, IDX_CHUNK), jnp.int32),
            pltpu.VMEM((2, IDX_CHUNK, 128), jnp.float32),
            pltpu.VMEM((2, OUT128_PER_STEP, 128), jnp.float32),
            pltpu.SemaphoreType.DMA,
            pltpu.SemaphoreType.DMA,
            pltpu.SemaphoreType.DMA,
        ],
    )
    def k(table_hbm, idx_hbm, out_hbm, idx_v, qoff_v, buf_v, rows_v,
          gsem, w0, w1):
        wid = lax.axis_index("s") * NC + lax.axis_index("c")
        out_base = wid * OUT128_PER_W
        pltpu.sync_copy(idx_hbm.at[wid], idx_v)

        def prep(c, carry):
            for v in range(8):
                vec = idx_v[c, pl.ds(v * 16, 16)]
                idx_v[c, pl.ds(v * 16, 16)] = lax.shift_right_logical(vec, 2)
                qoff_v[c, pl.ds(v * 16, 16)] = (vec & 3) * EMB
            return carry

        lax.fori_loop(0, IDX_ROWS, prep, 0)

        def fire(c, gbuf):
            return pltpu.async_copy(
                table_hbm.at[idx_v.at[c]], buf_v.at[gbuf], gsem)

        def wait_gather(c, gbuf):
            pltpu.make_async_copy(
                table_hbm.at[idx_v.at[c]], buf_v.at[gbuf], gsem).wait()

        def subsel(c, gbuf, rbuf, j):
            buf = buf_v.at[gbuf]
            dst_base = j * (IDX_CHUNK * EMB // 128)

            def body(g, carry):
                qvec = qoff_v[c, pl.ds(g * 16, 16)]
                for a in range(16):
                    q = qvec[a]
                    row = g * 16 + a
                    lo = buf[row, pl.ds(q, 16)]
                    hi = buf[row, pl.ds(q + 16, 16)]
                    r128 = dst_base + g * 4 + a // 4
                    rows_v[rbuf, r128, pl.ds((a % 4) * 32, 16)] = lo
                    rows_v[rbuf, r128, pl.ds((a % 4) * 32 + 16, 16)] = hi
                return carry

            lax.fori_loop(0, IDX_CHUNK // 16, body, 0)

        def write_step(s, rbuf, wsem_):
            return pltpu.async_copy(
                rows_v.at[rbuf],
                out_hbm.at[pl.ds(out_base + s * OUT128_PER_STEP,
                                 OUT128_PER_STEP)],
                wsem_)

        def drain_write(rbuf, wsem_):
            pltpu.make_async_copy(
                rows_v.at[rbuf],
                out_hbm.at[pl.ds(out_base, OUT128_PER_STEP)],
                wsem_).wait()

        def do_step(s, rbuf, wsem_, first):
            @pl.when(jnp.logical_not(first))
            def _():
                drain_write(rbuf, wsem_)

            for j in range(GATHERS_PER_STEP):
                c = s * GATHERS_PER_STEP + j
                wait_gather(c, j % 2)
                if j + 1 < GATHERS_PER_STEP:
                    fire(c + 1, (j + 1) % 2)
                else:
                    @pl.when(c + 1 < IDX_ROWS)
                    def _():
                        fire(c + 1, (j + 1) % 2)
                subsel(c, j % 2, rbuf, j)

            write_step(s, rbuf, wsem_)

        fire(0, 0)
        def pair(u, carry):
            do_step(2 * u, 0, w0, u == 0)
            do_step(2 * u + 1, 1, w1, u == 0)
            return carry

        lax.fori_loop(0, (STEPS - 1) // 2, pair, 0)
        do_step(STEPS - 1, 0, w0, jnp.bool_(False))
        drain_write(1, w1)
        drain_write(0, w0)

    return k(table128, idx)


def _mlp_body(h_ref, w1_ref, b1_ref, w2_ref, b2_ref, w3_ref, b3_ref, o_ref):
    h = h_ref[...]
    z = jnp.dot(h, w1_ref[...], preferred_element_type=jnp.float32)
    z = jnp.maximum(z + b1_ref[...], 0.0)
    z = jnp.dot(z, w2_ref[...], preferred_element_type=jnp.float32)
    z = jnp.maximum(z + b2_ref[...], 0.0)
    z = jnp.dot(z, w3_ref[...], preferred_element_type=jnp.float32)
    o_ref[...] = z + b3_ref[...]


def _mlp(h, W1, b1, W2, b2, W3, b3, block_b=1024):
    d_in = h.shape[1]
    grid = (BATCH // block_b,)
    return pl.pallas_call(
        _mlp_body,
        grid=grid,
        in_specs=[
            pl.BlockSpec((block_b, d_in), lambda i: (i, 0)),
            pl.BlockSpec((d_in, H1), lambda i: (0, 0)),
            pl.BlockSpec((1, H1), lambda i: (0, 0)),
            pl.BlockSpec((H1, H2), lambda i: (0, 0)),
            pl.BlockSpec((1, H2), lambda i: (0, 0)),
            pl.BlockSpec((H2, OUT), lambda i: (0, 0)),
            pl.BlockSpec((1, OUT), lambda i: (0, 0)),
        ],
        out_specs=pl.BlockSpec((block_b, OUT), lambda i: (i, 0)),
        out_shape=jax.ShapeDtypeStruct((BATCH, OUT), jnp.float32),
    )(h, W1, b1.reshape(1, H1), W2, b2.reshape(1, H2), W3,
      b3.reshape(1, OUT))


def kernel(x, table, W1, b1, W2, b2, W3, b3):
    idx = x.astype(jnp.int32).reshape(NW, IDX_ROWS, IDX_CHUNK)
    tail128 = table[NCH * 128:, :].reshape(16, 128)
    table128 = _table_convert(table.T, tail128)
    emb128 = _sc_gather(table128, idx)
    h = emb128.reshape(BATCH, N * EMB)
    return _mlp(h, W1, b1, W2, b2, W3, b3)


# speedup vs baseline: 1.4723x; 1.4723x over previous
import functools

import jax
import jax.numpy as jnp
from jax import lax
from jax.experimental import pallas as pl
from jax.experimental.pallas import tpu as pltpu
from jax.experimental.pallas import tpu_sc as plsc

N = 26
EMB = 32
OUT = 64
H1 = 256
H2 = 128
BATCH = 16384

NC = 2
NS = 16
NW = NC * NS

B_FLAT = BATCH * N
ROWS_PER_W = B_FLAT // NW
IDX_CHUNK = 128
IDX_ROWS = ROWS_PER_W // IDX_CHUNK
GATHERS_PER_STEP = 8
STEPS = IDX_ROWS // GATHERS_PER_STEP
STEP_ROWS = GATHERS_PER_STEP * IDX_CHUNK


def _sc_gather(table, idx):
    mesh = plsc.VectorSubcoreMesh(
        core_axis_name="c", subcore_axis_name="s", num_cores=NC,
        num_subcores=NS)

    @functools.partial(
        pl.kernel,
        out_type=jax.ShapeDtypeStruct((B_FLAT, EMB), jnp.float32),
        mesh=mesh,
        compiler_params=pltpu.CompilerParams(use_tc_tiling_on_sc=False),
        scratch_types=[
            pltpu.VMEM((IDX_ROWS, IDX_CHUNK), jnp.int32),
            pltpu.VMEM((2, STEP_ROWS, EMB), jnp.float32),
            pltpu.SemaphoreType.DMA,
            pltpu.SemaphoreType.DMA,
            pltpu.SemaphoreType.DMA,
            pltpu.SemaphoreType.DMA,
        ],
    )
    def k(table_hbm, idx_hbm, out_hbm, idx_v, rows_v, g0, g1, w0, w1):
        wid = lax.axis_index("s") * NC + lax.axis_index("c")
        base = wid * ROWS_PER_W
        gsem = (g0, g1)
        wsem = (w0, w1)
        pltpu.sync_copy(idx_hbm.at[wid], idx_v)

        def fire_gathers(step_idx, buf, sem):
            for j in range(GATHERS_PER_STEP):
                pltpu.async_copy(
                    table_hbm.at[idx_v.at[step_idx * GATHERS_PER_STEP + j]],
                    rows_v.at[buf, pl.ds(j * IDX_CHUNK, IDX_CHUNK)],
                    sem)

        def wait_bytes(buf, sem, out_off):
            pltpu.make_async_copy(
                rows_v.at[buf],
                out_hbm.at[pl.ds(out_off, STEP_ROWS)],
                sem).wait()

        fire_gathers(0, 0, gsem[0])
        for s in range(STEPS):
            buf = s % 2
            nxt = (s + 1) % 2
            if s + 1 < STEPS:
                if s >= 1:
                    wait_bytes(nxt, wsem[nxt], base)
                fire_gathers(s + 1, nxt, gsem[nxt])
            wait_bytes(buf, gsem[buf], base)
            pltpu.async_copy(rows_v.at[buf],
                             out_hbm.at[pl.ds(base + s * STEP_ROWS,
                                              STEP_ROWS)],
                             wsem[buf])
        wait_bytes(0, wsem[(STEPS - 1) % 2], base)
        wait_bytes(1, wsem[STEPS % 2], base)

    return k(table, idx)


def _mlp_body(h_ref, w1_ref, b1_ref, w2_ref, b2_ref, w3_ref, b3_ref, o_ref):
    h = h_ref[...]
    z = jnp.dot(h, w1_ref[...], preferred_element_type=jnp.float32)
    z = jnp.maximum(z + b1_ref[...], 0.0)
    z = jnp.dot(z, w2_ref[...], preferred_element_type=jnp.float32)
    z = jnp.maximum(z + b2_ref[...], 0.0)
    z = jnp.dot(z, w3_ref[...], preferred_element_type=jnp.float32)
    o_ref[...] = z + b3_ref[...]


def _mlp(h, W1, b1, W2, b2, W3, b3, block_b=1024):
    d_in = h.shape[1]
    grid = (BATCH // block_b,)
    return pl.pallas_call(
        _mlp_body,
        grid=grid,
        in_specs=[
            pl.BlockSpec((block_b, d_in), lambda i: (i, 0)),
            pl.BlockSpec((d_in, H1), lambda i: (0, 0)),
            pl.BlockSpec((1, H1), lambda i: (0, 0)),
            pl.BlockSpec((H1, H2), lambda i: (0, 0)),
            pl.BlockSpec((1, H2), lambda i: (0, 0)),
            pl.BlockSpec((H2, OUT), lambda i: (0, 0)),
            pl.BlockSpec((1, OUT), lambda i: (0, 0)),
        ],
        out_specs=pl.BlockSpec((block_b, OUT), lambda i: (i, 0)),
        out_shape=jax.ShapeDtypeStruct((BATCH, OUT), jnp.float32),
    )(h, W1, b1.reshape(1, H1), W2, b2.reshape(1, H2), W3,
      b3.reshape(1, OUT))


def kernel(x, table, W1, b1, W2, b2, W3, b3):
    idx = x.astype(jnp.int32).reshape(NW, IDX_ROWS, IDX_CHUNK)
    emb = _sc_gather(table, idx)
    h = emb.reshape(BATCH, N * EMB)
    return _mlp(h, W1, b1, W2, b2, W3, b3)
